# baseline (speedup 1.0000x reference)
import functools

import jax
import jax.numpy as jnp
from jax import lax
from jax.experimental import pallas as pl
from jax.experimental.pallas import tpu as pltpu
from jax.experimental.pallas import tpu_sc as plsc

B, N = 64, 1024
NN = N * N
CHUNK = 8192
C = NN // CHUNK
NBUF = 4
NC, NS = 2, 16
NW = NC * NS
SPW = B // NW
T = SPW * C


def _sc_body(hours_hbm, w_hbm, imf_hbm, out_hbm,
             hrs_v, w_v, in_buf, out_buf, in_sems, out_sems):
    wid = lax.axis_index("s") * NC + lax.axis_index("c")
    pltpu.sync_copy(hours_hbm, hrs_v)
    pltpu.sync_copy(w_hbm, w_v)
    b0 = wid * SPW

    def _hour(b):
        return hrs_v[pl.ds(b, 16)][0]

    def _weight(h):
        return jnp.maximum(w_v[pl.ds(h, 16)][0], 0.0)

    h0 = _hour(b0)
    for k in range(NBUF):
        pltpu.async_copy(imf_hbm.at[h0, pl.ds(k * CHUNK, CHUNK)],
                         in_buf.at[k], in_sems.at[k])

    @pl.loop(0, T, step=NBUF)
    def _chunks(g):
        for k in range(NBUF):
            t = g + k
            b = b0 + t // C
            c = t % C
            h = _hour(b)
            wv = _weight(h)

            pltpu.make_async_copy(imf_hbm.at[h, pl.ds(c * CHUNK, CHUNK)],
                                  in_buf.at[k], in_sems.at[k]).wait()

            @pl.when(g > 0)
            def _drain_out():
                pltpu.make_async_copy(out_buf.at[k],
                                      out_hbm.at[b, pl.ds(c * CHUNK, CHUNK)],
                                      out_sems.at[k]).wait()

            @pl.loop(0, CHUNK // 16, unroll=8)
            def _scale(j):
                sl = pl.ds(j * 16, 16)
                out_buf[k, sl] = in_buf[k, sl] * wv

            pltpu.async_copy(out_buf.at[k],
                             out_hbm.at[b, pl.ds(c * CHUNK, CHUNK)],
                             out_sems.at[k])

            @pl.when(t + NBUF < T)
            def _prefetch():
                t2 = t + NBUF
                b2 = b0 + t2 // C
                c2 = t2 % C
                h2 = _hour(b2)
                pltpu.async_copy(imf_hbm.at[h2, pl.ds(c2 * CHUNK, CHUNK)],
                                 in_buf.at[k], in_sems.at[k])

    for k in range(NBUF):
        pltpu.make_async_copy(out_buf.at[k], out_hbm.at[0, pl.ds(0, CHUNK)],
                              out_sems.at[k]).wait()


_sc_call = functools.partial(
    pl.kernel,
    out_type=jax.ShapeDtypeStruct((B, NN), jnp.float32),
    mesh=plsc.VectorSubcoreMesh(core_axis_name="c", subcore_axis_name="s"),
    scratch_types=[
        pltpu.VMEM((B + 16,), jnp.int32),
        pltpu.VMEM((40,), jnp.float32),
        pltpu.VMEM((NBUF, CHUNK), jnp.float32),
        pltpu.VMEM((NBUF, CHUNK), jnp.float32),
        pltpu.SemaphoreType.DMA((NBUF,)),
        pltpu.SemaphoreType.DMA((NBUF,)),
    ],
)(_sc_body)


def kernel(inputs, imf, weights):
    hours = (inputs[:, 0, 0, 1] * 24.0).astype(jnp.int32)
    hours_pad = jnp.pad(hours, (0, 16))
    w_pad = jnp.pad(weights.reshape(24), (0, 16))
    out = _sc_call(hours_pad, w_pad, imf.reshape(24, NN))
    return out.reshape(B, N, N)


# speedup vs baseline: 1.9814x; 1.9814x over previous
import functools

import jax
import jax.numpy as jnp
from jax import lax
from jax.experimental import pallas as pl
from jax.experimental.pallas import tpu as pltpu
from jax.experimental.pallas import tpu_sc as plsc

B, N = 64, 1024
NN = N * N
CHUNK = 8192
C = NN // CHUNK
NBUF = 4
NC, NS = 2, 16
NW = NC * NS
SPW = B // NW
T = SPW * C


def _sc_body(hours_hbm, w_hbm, imf_hbm, out_hbm,
             hrs_v, w_v, in_buf, out_buf, in_sems, out_sems):
    wid = lax.axis_index("s") * NC + lax.axis_index("c")
    pltpu.sync_copy(hours_hbm, hrs_v)
    pltpu.sync_copy(w_hbm, w_v)
    b0 = wid * SPW

    def _hour(b):
        return hrs_v[pl.ds(b, 16)][0]

    def _weight(h):
        return jnp.maximum(w_v[pl.ds(h, 16)][0], 0.0)

    h0 = _hour(b0)
    for k in range(NBUF):
        pltpu.async_copy(imf_hbm.at[h0, pl.ds(k * CHUNK, CHUNK)],
                         in_buf.at[k], in_sems.at[k])

    @pl.loop(0, T, step=NBUF)
    def _chunks(g):
        for k in range(NBUF):
            t = g + k
            b = b0 + t // C
            c = t % C
            h = _hour(b)
            wv = _weight(h)

            pltpu.make_async_copy(imf_hbm.at[h, pl.ds(c * CHUNK, CHUNK)],
                                  in_buf.at[k], in_sems.at[k]).wait()

            @pl.when(g > 0)
            def _drain_out():
                pltpu.make_async_copy(out_buf.at[k],
                                      out_hbm.at[b, pl.ds(c * CHUNK, CHUNK)],
                                      out_sems.at[k]).wait()

            @plsc.parallel_loop(0, CHUNK, step=16, unroll=8)
            def _scale(j):
                sl = pl.ds(j, 16)
                out_buf[k, sl] = in_buf[k, sl] * wv

            pltpu.async_copy(out_buf.at[k],
                             out_hbm.at[b, pl.ds(c * CHUNK, CHUNK)],
                             out_sems.at[k])

            @pl.when(t + NBUF < T)
            def _prefetch():
                t2 = t + NBUF
                b2 = b0 + t2 // C
                c2 = t2 % C
                h2 = _hour(b2)
                pltpu.async_copy(imf_hbm.at[h2, pl.ds(c2 * CHUNK, CHUNK)],
                                 in_buf.at[k], in_sems.at[k])

    for k in range(NBUF):
        pltpu.make_async_copy(out_buf.at[k], out_hbm.at[0, pl.ds(0, CHUNK)],
                              out_sems.at[k]).wait()


_sc_call = functools.partial(
    pl.kernel,
    out_type=jax.ShapeDtypeStruct((B, NN), jnp.float32),
    mesh=plsc.VectorSubcoreMesh(core_axis_name="c", subcore_axis_name="s"),
    scratch_types=[
        pltpu.VMEM((B + 16,), jnp.int32),
        pltpu.VMEM((40,), jnp.float32),
        pltpu.VMEM((NBUF, CHUNK), jnp.float32),
        pltpu.VMEM((NBUF, CHUNK), jnp.float32),
        pltpu.SemaphoreType.DMA((NBUF,)),
        pltpu.SemaphoreType.DMA((NBUF,)),
    ],
)(_sc_body)


def kernel(inputs, imf, weights):
    hours = (inputs[:, 0, 0, 1] * 24.0).astype(jnp.int32)
    hours_pad = jnp.pad(hours, (0, 16))
    w_pad = jnp.pad(weights.reshape(24), (0, 16))
    out = _sc_call(hours_pad, w_pad, imf.reshape(24, NN))
    return out.reshape(B, N, N)


# speedup vs baseline: 2.3973x; 1.2099x over previous
import functools

import jax
import jax.numpy as jnp
from jax import lax
from jax.experimental import pallas as pl
from jax.experimental.pallas import tpu as pltpu
from jax.experimental.pallas import tpu_sc as plsc

B, N = 64, 1024
NN = N * N
NH = 24
CHUNK = 8192
C = NN // CHUNK
NC, NS = 2, 16
NW = NC * NS
CCW = C // NW


def _sc_body(cnt_hbm, start_hbm, order_hbm, w_hbm, imf_hbm, out_hbm,
             cnt_v, start_v, order_v, w_v, in_buf, out_buf,
             in_sems, out_sems, prev_smem):
    wid = lax.axis_index("s") * NC + lax.axis_index("c")
    pltpu.sync_copy(cnt_hbm, cnt_v)
    pltpu.sync_copy(start_hbm, start_v)
    pltpu.sync_copy(order_hbm, order_v)
    pltpu.sync_copy(w_hbm, w_v)
    for cc in range(CCW):
        prev_smem[cc] = 0

    def _sget(ref, i):
        return ref[pl.ds(i, 16)][0]

    def read_chunk(h, hh, cc):
        c = cc * NW + wid
        pltpu.async_copy(imf_hbm.at[h, pl.ds(c * CHUNK, CHUNK)],
                         in_buf.at[hh * CCW + cc], in_sems.at[hh * CCW + cc])

    for hh in range(2):
        for cc in range(CCW):
            read_chunk(hh, hh, cc)

    @pl.loop(0, NH, step=2)
    def _hloop(g):
        for hh in range(2):
            h = g + hh
            wv = jnp.maximum(_sget(w_v, h), 0.0)
            cnt = _sget(cnt_v, h)
            st = _sget(start_v, h)
            for cc in range(CCW):
                slot = hh * CCW + cc
                c = cc * NW + wid

                pltpu.make_async_copy(imf_hbm.at[0, pl.ds(0, CHUNK)],
                                      in_buf.at[slot], in_sems.at[slot]).wait()

                prev = prev_smem[cc]

                @pl.loop(0, prev)
                def _drain(j):
                    pltpu.make_async_copy(out_buf.at[cc],
                                          out_hbm.at[0, pl.ds(0, CHUNK)],
                                          out_sems.at[cc]).wait()

                @plsc.parallel_loop(0, CHUNK, step=16, unroll=16)
                def _scale(j):
                    sl = pl.ds(j, 16)
                    out_buf[cc, sl] = in_buf[slot, sl] * wv

                @pl.when(h + 2 < NH)
                def _prefetch():
                    read_chunk(h + 2, hh, cc)

                @pl.loop(0, cnt)
                def _writes(j):
                    b = _sget(order_v, st + j)
                    pltpu.async_copy(out_buf.at[cc],
                                     out_hbm.at[b, pl.ds(c * CHUNK, CHUNK)],
                                     out_sems.at[cc])

                prev_smem[cc] = cnt

    for cc in range(CCW):
        prev = prev_smem[cc]

        @pl.loop(0, prev)
        def _final_drain(j):
            pltpu.make_async_copy(out_buf.at[cc],
                                  out_hbm.at[0, pl.ds(0, CHUNK)],
                                  out_sems.at[cc]).wait()


_sc_call = functools.partial(
    pl.kernel,
    out_type=jax.ShapeDtypeStruct((B, NN), jnp.float32),
    mesh=plsc.VectorSubcoreMesh(core_axis_name="c", subcore_axis_name="s"),
    scratch_types=[
        pltpu.VMEM((NH + 16,), jnp.int32),
        pltpu.VMEM((NH + 16,), jnp.int32),
        pltpu.VMEM((B + 16,), jnp.int32),
        pltpu.VMEM((NH + 16,), jnp.float32),
        pltpu.VMEM((2 * CCW, CHUNK), jnp.float32),
        pltpu.VMEM((CCW, CHUNK), jnp.float32),
        pltpu.SemaphoreType.DMA((2 * CCW,)),
        pltpu.SemaphoreType.DMA((CCW,)),
        pltpu.SMEM((CCW,), jnp.int32),
    ],
)(_sc_body)


def kernel(inputs, imf, weights):
    hours = (inputs[:, 0, 0, 1] * 24.0).astype(jnp.int32)
    order = jnp.argsort(hours).astype(jnp.int32)
    cnt = jnp.bincount(hours, length=NH).astype(jnp.int32)
    start = (jnp.cumsum(cnt) - cnt).astype(jnp.int32)
    out = _sc_call(
        jnp.pad(cnt, (0, 16)),
        jnp.pad(start, (0, 16)),
        jnp.pad(order, (0, 16)),
        jnp.pad(weights.reshape(NH), (0, 16)),
        imf.reshape(NH, NN),
    )
    return out.reshape(B, N, N)


# speedup vs baseline: 6.1872x; 2.5809x over previous
import functools

import jax
import jax.numpy as jnp
from jax import lax
from jax.experimental import pallas as pl
from jax.experimental.pallas import tpu as pltpu
from jax.experimental.pallas import tpu_sc as plsc

B, N = 64, 1024
NH = 24
RPC = 8
C = N // RPC
NC, NS = 2, 16
NW = NC * NS
CCW = C // NW


def _sc_body(cnt_hbm, start_hbm, order_hbm, w_hbm, imf_hbm, out_hbm,
             cnt_v, start_v, order_v, w_v, in_buf, out_buf,
             in_sems, out_sems, prev_smem):
    wid = lax.axis_index("s") * NC + lax.axis_index("c")
    pltpu.sync_copy(cnt_hbm, cnt_v)
    pltpu.sync_copy(start_hbm, start_v)
    pltpu.sync_copy(order_hbm, order_v)
    pltpu.sync_copy(w_hbm, w_v)
    for cc in range(CCW):
        prev_smem[cc] = 0

    def _sget(ref, i):
        return ref[pl.ds(i, 16)][0]

    def read_chunk(h, hh, cc):
        r0 = (cc * NW + wid) * RPC
        pltpu.async_copy(imf_hbm.at[h, pl.ds(r0, RPC), :],
                         in_buf.at[hh * CCW + cc], in_sems.at[hh * CCW + cc])

    for hh in range(2):
        for cc in range(CCW):
            read_chunk(hh, hh, cc)

    @pl.loop(0, NH, step=2)
    def _hloop(g):
        for hh in range(2):
            h = g + hh
            wv = jnp.maximum(_sget(w_v, h), 0.0)
            cnt = _sget(cnt_v, h)
            st = _sget(start_v, h)
            for cc in range(CCW):
                slot = hh * CCW + cc
                r0 = (cc * NW + wid) * RPC

                pltpu.make_async_copy(imf_hbm.at[0, pl.ds(0, RPC), :],
                                      in_buf.at[slot], in_sems.at[slot]).wait()

                prev = prev_smem[cc]

                @pl.loop(0, prev)
                def _drain(j):
                    pltpu.make_async_copy(out_buf.at[cc],
                                          out_hbm.at[0, pl.ds(0, RPC), :],
                                          out_sems.at[cc]).wait()

                for r in range(RPC):

                    @plsc.parallel_loop(0, N, step=16, unroll=16)
                    def _scale(j):
                        sl = pl.ds(j, 16)
                        out_buf[cc, r, sl] = in_buf[slot, r, sl] * wv

                @pl.when(h + 2 < NH)
                def _prefetch():
                    read_chunk(h + 2, hh, cc)

                @pl.loop(0, cnt)
                def _writes(j):
                    b = _sget(order_v, st + j)
                    pltpu.async_copy(out_buf.at[cc],
                                     out_hbm.at[b, pl.ds(r0, RPC), :],
                                     out_sems.at[cc])

                prev_smem[cc] = cnt

    for cc in range(CCW):
        prev = prev_smem[cc]

        @pl.loop(0, prev)
        def _final_drain(j):
            pltpu.make_async_copy(out_buf.at[cc],
                                  out_hbm.at[0, pl.ds(0, RPC), :],
                                  out_sems.at[cc]).wait()


_sc_call = functools.partial(
    pl.kernel,
    out_type=jax.ShapeDtypeStruct((B, N, N), jnp.float32),
    mesh=plsc.VectorSubcoreMesh(core_axis_name="c", subcore_axis_name="s"),
    scratch_types=[
        pltpu.VMEM((NH + 16,), jnp.int32),
        pltpu.VMEM((NH + 16,), jnp.int32),
        pltpu.VMEM((B + 16,), jnp.int32),
        pltpu.VMEM((NH + 16,), jnp.float32),
        pltpu.VMEM((2 * CCW, RPC, N), jnp.float32),
        pltpu.VMEM((CCW, RPC, N), jnp.float32),
        pltpu.SemaphoreType.DMA((2 * CCW,)),
        pltpu.SemaphoreType.DMA((CCW,)),
        pltpu.SMEM((CCW,), jnp.int32),
    ],
)(_sc_body)


def kernel(inputs, imf, weights):
    hours = (inputs[:, 0, 0, 1] * 24.0).astype(jnp.int32)
    order = jnp.argsort(hours).astype(jnp.int32)
    cnt = jnp.bincount(hours, length=NH).astype(jnp.int32)
    start = (jnp.cumsum(cnt) - cnt).astype(jnp.int32)
    return _sc_call(
        jnp.pad(cnt, (0, 16)),
        jnp.pad(start, (0, 16)),
        jnp.pad(order, (0, 16)),
        jnp.pad(weights.reshape(NH), (0, 16)),
        imf,
    )
